# baseline (device time: 91816 ns/iter reference)
import jax
import jax.numpy as jnp
from jax import lax
from jax.experimental import pallas as pl
from jax.experimental.pallas import tpu as pltpu

N_DEV = 16
SQ = 1024
SKV = 1024
D_MODEL = 1024
HALF_D = D_MODEL // 2
HEADS_PER_SHARD = 8
DH = 128
WINDOW = 128
SCALE = 0.08838834764831843

MASKS_A = (1, 2, 4, 8)
MASKS_B = (4, 8, 2, 1)
CONTRIB_A = {1: 512, 2: 256, 4: 128, 8: 64}
CONTRIB_B = {4: 512, 8: 256, 2: 128, 1: 64}


def _body(x_ref, wq_ref, k_ref, v_ref, wo_ref, out_ref,
          q_ref, ctx_ref, acc_ref,
          accb_ref, rbufa_ref, rbufb_ref, gbufa_ref, gbufb_ref,
          rsa_send, rsa_recv, rsb_send, rsb_recv,
          dba_send, dba_recv, dbb_send, dbb_recv):
    my = lax.axis_index("i")

    q = lax.dot_general(
        x_ref[...], wq_ref[...], (((1,), (0,)), ((), ())),
        preferred_element_type=jnp.float32,
    )
    q_ref[...] = q.astype(jnp.bfloat16)

    RBLK = 256
    for h in range(HEADS_PER_SHARD):
        for r in range(SQ // RBLK):
            r0 = r * RBLK
            c0 = max(0, r0 - WINDOW)
            c1 = min(SKV, r0 + RBLK + WINDOW)
            w = c1 - c0
            qblk = q_ref[r0:r0 + RBLK, h * DH:(h + 1) * DH]
            scores = lax.dot_general(
                qblk, k_ref[c0:c1, h, :], (((1,), (1,)), ((), ())),
                preferred_element_type=jnp.float32,
            ) * SCALE
            rows = lax.broadcasted_iota(jnp.int32, (RBLK, w), 0) + r0
            cols = lax.broadcasted_iota(jnp.int32, (RBLK, w), 1) + c0
            scores = jnp.where(jnp.abs(rows - cols) <= WINDOW, scores, -1e9)
            m = jnp.max(scores, axis=1, keepdims=True)
            e = jnp.exp(scores - m)
            s = jnp.sum(e, axis=1, keepdims=True)
            wgt = (e / s).astype(jnp.bfloat16)
            ctx = lax.dot_general(
                wgt, v_ref[c0:c1, h, :], (((1,), (0,)), ((), ())),
                preferred_element_type=jnp.float32,
            )
            ctx_ref[r0:r0 + RBLK, h * DH:(h + 1) * DH] = ctx.astype(
                jnp.bfloat16
            )

    accv = lax.dot_general(
        ctx_ref[...], wo_ref[...], (((1,), (0,)), ((), ())),
        preferred_element_type=jnp.float32,
    )
    acc_ref[...] = accv
    accb_ref[...] = accv.astype(jnp.bfloat16)

    bsem = pltpu.get_barrier_semaphore()
    for mk in MASKS_A:
        pl.semaphore_signal(bsem, inc=1, device_id=(my ^ mk,),
                            device_id_type=pl.DeviceIdType.MESH)
    pl.semaphore_wait(bsem, 4)

    def _keep_give(start, half, mk):
        upper = (my & mk) != 0
        keep = pl.multiple_of(
            jnp.where(upper, start + half, start).astype(jnp.int32), 64
        )
        give = pl.multiple_of(
            jnp.where(upper, start, start + half).astype(jnp.int32), 64
        )
        return keep, give

    start_a = jnp.int32(0)
    start_b = jnp.int32(0)
    length = SQ
    for k in range(4):
        half = length // 2
        mka, mkb = MASKS_A[k], MASKS_B[k]
        keep_a, give_a = _keep_give(start_a, half, mka)
        keep_b, give_b = _keep_give(start_b, half, mkb)
        step_a = pltpu.make_async_remote_copy(
            src_ref=accb_ref.at[pl.ds(give_a, half), 0:HALF_D],
            dst_ref=rbufa_ref.at[k, 0:half, :],
            send_sem=rsa_send.at[k], recv_sem=rsa_recv.at[k],
            device_id=(my ^ mka,), device_id_type=pl.DeviceIdType.MESH,
        )
        step_b = pltpu.make_async_remote_copy(
            src_ref=accb_ref.at[pl.ds(give_b, half), HALF_D:D_MODEL],
            dst_ref=rbufb_ref.at[k, 0:half, :],
            send_sem=rsb_send.at[k], recv_sem=rsb_recv.at[k],
            device_id=(my ^ mkb,), device_id_type=pl.DeviceIdType.MESH,
        )
        step_a.start()
        step_b.start()
        step_a.wait_send()
        step_a.wait_recv()
        new_a = (
            acc_ref[pl.ds(keep_a, half), 0:HALF_D]
            + rbufa_ref[k, 0:half, :].astype(jnp.float32)
        )
        acc_ref[pl.ds(keep_a, half), 0:HALF_D] = new_a
        accb_ref[pl.ds(keep_a, half), 0:HALF_D] = new_a.astype(jnp.bfloat16)
        step_b.wait_send()
        step_b.wait_recv()
        new_b = (
            acc_ref[pl.ds(keep_b, half), HALF_D:D_MODEL]
            + rbufb_ref[k, 0:half, :].astype(jnp.float32)
        )
        acc_ref[pl.ds(keep_b, half), HALF_D:D_MODEL] = new_b
        accb_ref[pl.ds(keep_b, half), HALF_D:D_MODEL] = new_b.astype(
            jnp.bfloat16
        )
        start_a, start_b = keep_a, keep_b
        length = half

    gbufa_ref[pl.ds(start_a, 64), :] = accb_ref[pl.ds(start_a, 64), 0:HALF_D]
    gbufb_ref[pl.ds(start_b, 64), :] = accb_ref[
        pl.ds(start_b, 64), HALF_D:D_MODEL]
    cur_a, cur_b = start_a, start_b
    cur_len = 64
    for k in range(4):
        mka = MASKS_A[3 - k]
        mkb = MASKS_B[3 - k]
        pstart_a = pl.multiple_of(
            jnp.where((my & mka) != 0, cur_a - CONTRIB_A[mka],
                      cur_a + CONTRIB_A[mka]).astype(jnp.int32), 64)
        pstart_b = pl.multiple_of(
            jnp.where((my & mkb) != 0, cur_b - CONTRIB_B[mkb],
                      cur_b + CONTRIB_B[mkb]).astype(jnp.int32), 64)
        send_a = pltpu.make_async_remote_copy(
            src_ref=gbufa_ref.at[pl.ds(cur_a, cur_len), :],
            dst_ref=gbufa_ref.at[pl.ds(cur_a, cur_len), :],
            send_sem=dba_send.at[k], recv_sem=dba_recv.at[k],
            device_id=(my ^ mka,), device_id_type=pl.DeviceIdType.MESH,
        )
        send_b = pltpu.make_async_remote_copy(
            src_ref=gbufb_ref.at[pl.ds(cur_b, cur_len), :],
            dst_ref=gbufb_ref.at[pl.ds(cur_b, cur_len), :],
            send_sem=dbb_send.at[k], recv_sem=dbb_recv.at[k],
            device_id=(my ^ mkb,), device_id_type=pl.DeviceIdType.MESH,
        )
        send_a.start()
        send_b.start()
        send_a.wait_send()
        recv_a = pltpu.make_async_remote_copy(
            src_ref=gbufa_ref.at[pl.ds(pstart_a, cur_len), :],
            dst_ref=gbufa_ref.at[pl.ds(pstart_a, cur_len), :],
            send_sem=dba_send.at[k], recv_sem=dba_recv.at[k],
            device_id=(my ^ mka,), device_id_type=pl.DeviceIdType.MESH,
        )
        recv_a.wait_recv()
        send_b.wait_send()
        recv_b = pltpu.make_async_remote_copy(
            src_ref=gbufb_ref.at[pl.ds(pstart_b, cur_len), :],
            dst_ref=gbufb_ref.at[pl.ds(pstart_b, cur_len), :],
            send_sem=dbb_send.at[k], recv_sem=dbb_recv.at[k],
            device_id=(my ^ mkb,), device_id_type=pl.DeviceIdType.MESH,
        )
        recv_b.wait_recv()
        cur_a = pl.multiple_of(jnp.minimum(cur_a, pstart_a), 64)
        cur_b = pl.multiple_of(jnp.minimum(cur_b, pstart_b), 64)
        cur_len *= 2

    out_ref[:, 0:HALF_D] = gbufa_ref[...].astype(jnp.float32)
    out_ref[:, HALF_D:D_MODEL] = gbufb_ref[...].astype(jnp.float32)


def kernel(x, Wq, K_ext, V_ext, Wo):
    pos = lax.axis_index("i")
    xb = x[0].astype(jnp.bfloat16)
    wq = Wq.astype(jnp.bfloat16)
    wo = Wo.astype(jnp.bfloat16)
    kh = lax.dynamic_slice(
        K_ext, (0, 0, pos * HEADS_PER_SHARD, 0), (1, SKV, HEADS_PER_SHARD, DH)
    )[0]
    vh = lax.dynamic_slice(
        V_ext, (0, 0, pos * HEADS_PER_SHARD, 0), (1, SKV, HEADS_PER_SHARD, DH)
    )[0]
    kh = kh.astype(jnp.bfloat16)
    vh = vh.astype(jnp.bfloat16)

    out = pl.pallas_call(
        _body,
        out_shape=jax.ShapeDtypeStruct((SQ, D_MODEL), jnp.float32),
        in_specs=[pl.BlockSpec(memory_space=pltpu.VMEM)] * 5,
        out_specs=pl.BlockSpec(memory_space=pltpu.VMEM),
        scratch_shapes=[
            pltpu.VMEM((SQ, D_MODEL), jnp.bfloat16),
            pltpu.VMEM((SQ, D_MODEL), jnp.bfloat16),
            pltpu.VMEM((SQ, D_MODEL), jnp.float32),
            pltpu.VMEM((SQ, D_MODEL), jnp.bfloat16),
            pltpu.VMEM((4, SQ // 2, HALF_D), jnp.bfloat16),
            pltpu.VMEM((4, SQ // 2, HALF_D), jnp.bfloat16),
            pltpu.VMEM((SQ, HALF_D), jnp.bfloat16),
            pltpu.VMEM((SQ, HALF_D), jnp.bfloat16),
            pltpu.SemaphoreType.DMA((4,)),
            pltpu.SemaphoreType.DMA((4,)),
            pltpu.SemaphoreType.DMA((4,)),
            pltpu.SemaphoreType.DMA((4,)),
            pltpu.SemaphoreType.DMA((4,)),
            pltpu.SemaphoreType.DMA((4,)),
            pltpu.SemaphoreType.DMA((4,)),
            pltpu.SemaphoreType.DMA((4,)),
        ],
        compiler_params=pltpu.CompilerParams(collective_id=0),
    )(xb, wq, kh, vh, wo)
    return out.reshape(1, SQ, D_MODEL)


# device time: 82662 ns/iter; 1.1107x vs baseline; 1.1107x over previous
import jax
import jax.numpy as jnp
from jax import lax
from jax.experimental import pallas as pl
from jax.experimental.pallas import tpu as pltpu

N_DEV = 16
SQ = 1024
SKV = 1024
D_MODEL = 1024
HALF_D = D_MODEL // 2
HEADS_PER_SHARD = 8
DH = 128
WINDOW = 128
SCALE = 0.08838834764831843

MASKS_A = (1, 2, 4, 8)
MASKS_B = (4, 8, 2, 1)
CONTRIB_A = {1: 512, 2: 256, 4: 128, 8: 64}
CONTRIB_B = {4: 512, 8: 256, 2: 128, 1: 64}


def _body(x_ref, wq_ref, k_ref, v_ref, wo_ref, out_ref,
          q_ref, ctx_ref, acc_ref,
          accb_ref, rbufa_ref, rbufb_ref, gbufa_ref, gbufb_ref,
          rsa_send, rsa_recv, rsb_send, rsb_recv,
          dba_send, dba_recv, dbb_send, dbb_recv):
    my = lax.axis_index("i")

    q = lax.dot_general(
        x_ref[...], wq_ref[...], (((1,), (0,)), ((), ())),
        preferred_element_type=jnp.float32,
    )
    q_ref[...] = q.astype(jnp.bfloat16)

    RBLK = 256
    for h in range(HEADS_PER_SHARD):
        for r in range(SQ // RBLK):
            r0 = r * RBLK
            c0 = max(0, r0 - WINDOW)
            c1 = min(SKV, r0 + RBLK + WINDOW)
            w = c1 - c0
            qblk = q_ref[r0:r0 + RBLK, h * DH:(h + 1) * DH]
            scores = lax.dot_general(
                qblk, k_ref[h, c0:c1, :], (((1,), (1,)), ((), ())),
                preferred_element_type=jnp.float32,
            ) * SCALE
            rows = lax.broadcasted_iota(jnp.int32, (RBLK, w), 0) + r0
            cols = lax.broadcasted_iota(jnp.int32, (RBLK, w), 1) + c0
            scores = jnp.where(jnp.abs(rows - cols) <= WINDOW, scores, -1e9)
            m = jnp.max(scores, axis=1, keepdims=True)
            e = jnp.exp(scores - m)
            s = jnp.sum(e, axis=1, keepdims=True)
            wgt = (e / s).astype(jnp.bfloat16)
            ctx = lax.dot_general(
                wgt, v_ref[h, c0:c1, :], (((1,), (0,)), ((), ())),
                preferred_element_type=jnp.float32,
            )
            ctx_ref[r0:r0 + RBLK, h * DH:(h + 1) * DH] = ctx.astype(
                jnp.bfloat16
            )

    accv = lax.dot_general(
        ctx_ref[...], wo_ref[...], (((1,), (0,)), ((), ())),
        preferred_element_type=jnp.float32,
    )
    acc_ref[...] = accv
    accb_ref[...] = accv.astype(jnp.bfloat16)

    bsem = pltpu.get_barrier_semaphore()
    for mk in MASKS_A:
        pl.semaphore_signal(bsem, inc=1, device_id=(my ^ mk,),
                            device_id_type=pl.DeviceIdType.MESH)
    pl.semaphore_wait(bsem, 4)

    def _keep_give(start, half, mk):
        upper = (my & mk) != 0
        keep = pl.multiple_of(
            jnp.where(upper, start + half, start).astype(jnp.int32), 64
        )
        give = pl.multiple_of(
            jnp.where(upper, start, start + half).astype(jnp.int32), 64
        )
        return keep, give

    start_a = jnp.int32(0)
    start_b = jnp.int32(0)
    length = SQ
    for k in range(4):
        half = length // 2
        mka, mkb = MASKS_A[k], MASKS_B[k]
        keep_a, give_a = _keep_give(start_a, half, mka)
        keep_b, give_b = _keep_give(start_b, half, mkb)
        step_a = pltpu.make_async_remote_copy(
            src_ref=accb_ref.at[pl.ds(give_a, half), 0:HALF_D],
            dst_ref=rbufa_ref.at[k, 0:half, :],
            send_sem=rsa_send.at[k], recv_sem=rsa_recv.at[k],
            device_id=(my ^ mka,), device_id_type=pl.DeviceIdType.MESH,
        )
        step_b = pltpu.make_async_remote_copy(
            src_ref=accb_ref.at[pl.ds(give_b, half), HALF_D:D_MODEL],
            dst_ref=rbufb_ref.at[k, 0:half, :],
            send_sem=rsb_send.at[k], recv_sem=rsb_recv.at[k],
            device_id=(my ^ mkb,), device_id_type=pl.DeviceIdType.MESH,
        )
        step_a.start()
        step_b.start()
        step_a.wait_send()
        step_a.wait_recv()
        new_a = (
            acc_ref[pl.ds(keep_a, half), 0:HALF_D]
            + rbufa_ref[k, 0:half, :].astype(jnp.float32)
        )
        acc_ref[pl.ds(keep_a, half), 0:HALF_D] = new_a
        accb_ref[pl.ds(keep_a, half), 0:HALF_D] = new_a.astype(jnp.bfloat16)
        step_b.wait_send()
        step_b.wait_recv()
        new_b = (
            acc_ref[pl.ds(keep_b, half), HALF_D:D_MODEL]
            + rbufb_ref[k, 0:half, :].astype(jnp.float32)
        )
        acc_ref[pl.ds(keep_b, half), HALF_D:D_MODEL] = new_b
        accb_ref[pl.ds(keep_b, half), HALF_D:D_MODEL] = new_b.astype(
            jnp.bfloat16
        )
        start_a, start_b = keep_a, keep_b
        length = half

    gbufa_ref[pl.ds(start_a, 64), :] = accb_ref[pl.ds(start_a, 64), 0:HALF_D]
    gbufb_ref[pl.ds(start_b, 64), :] = accb_ref[
        pl.ds(start_b, 64), HALF_D:D_MODEL]
    cur_a, cur_b = start_a, start_b
    cur_len = 64
    for k in range(4):
        mka = MASKS_A[3 - k]
        mkb = MASKS_B[3 - k]
        pstart_a = pl.multiple_of(
            jnp.where((my & mka) != 0, cur_a - CONTRIB_A[mka],
                      cur_a + CONTRIB_A[mka]).astype(jnp.int32), 64)
        pstart_b = pl.multiple_of(
            jnp.where((my & mkb) != 0, cur_b - CONTRIB_B[mkb],
                      cur_b + CONTRIB_B[mkb]).astype(jnp.int32), 64)
        send_a = pltpu.make_async_remote_copy(
            src_ref=gbufa_ref.at[pl.ds(cur_a, cur_len), :],
            dst_ref=gbufa_ref.at[pl.ds(cur_a, cur_len), :],
            send_sem=dba_send.at[k], recv_sem=dba_recv.at[k],
            device_id=(my ^ mka,), device_id_type=pl.DeviceIdType.MESH,
        )
        send_b = pltpu.make_async_remote_copy(
            src_ref=gbufb_ref.at[pl.ds(cur_b, cur_len), :],
            dst_ref=gbufb_ref.at[pl.ds(cur_b, cur_len), :],
            send_sem=dbb_send.at[k], recv_sem=dbb_recv.at[k],
            device_id=(my ^ mkb,), device_id_type=pl.DeviceIdType.MESH,
        )
        send_a.start()
        send_b.start()
        send_a.wait_send()
        recv_a = pltpu.make_async_remote_copy(
            src_ref=gbufa_ref.at[pl.ds(pstart_a, cur_len), :],
            dst_ref=gbufa_ref.at[pl.ds(pstart_a, cur_len), :],
            send_sem=dba_send.at[k], recv_sem=dba_recv.at[k],
            device_id=(my ^ mka,), device_id_type=pl.DeviceIdType.MESH,
        )
        recv_a.wait_recv()
        send_b.wait_send()
        recv_b = pltpu.make_async_remote_copy(
            src_ref=gbufb_ref.at[pl.ds(pstart_b, cur_len), :],
            dst_ref=gbufb_ref.at[pl.ds(pstart_b, cur_len), :],
            send_sem=dbb_send.at[k], recv_sem=dbb_recv.at[k],
            device_id=(my ^ mkb,), device_id_type=pl.DeviceIdType.MESH,
        )
        recv_b.wait_recv()
        cur_a = pl.multiple_of(jnp.minimum(cur_a, pstart_a), 64)
        cur_b = pl.multiple_of(jnp.minimum(cur_b, pstart_b), 64)
        cur_len *= 2

    out_ref[:, 0:HALF_D] = gbufa_ref[...].astype(jnp.float32)
    out_ref[:, HALF_D:D_MODEL] = gbufb_ref[...].astype(jnp.float32)


def kernel(x, Wq, K_ext, V_ext, Wo):
    pos = lax.axis_index("i")
    xb = x[0].astype(jnp.bfloat16)
    wq = Wq.astype(jnp.bfloat16)
    wo = Wo.astype(jnp.bfloat16)
    kh = lax.dynamic_slice(
        K_ext, (0, 0, pos * HEADS_PER_SHARD, 0), (1, SKV, HEADS_PER_SHARD, DH)
    )[0]
    vh = lax.dynamic_slice(
        V_ext, (0, 0, pos * HEADS_PER_SHARD, 0), (1, SKV, HEADS_PER_SHARD, DH)
    )[0]
    kh = jnp.transpose(kh, (1, 0, 2)).astype(jnp.bfloat16)
    vh = jnp.transpose(vh, (1, 0, 2)).astype(jnp.bfloat16)

    out = pl.pallas_call(
        _body,
        out_shape=jax.ShapeDtypeStruct((SQ, D_MODEL), jnp.float32),
        in_specs=[pl.BlockSpec(memory_space=pltpu.VMEM)] * 5,
        out_specs=pl.BlockSpec(memory_space=pltpu.VMEM),
        scratch_shapes=[
            pltpu.VMEM((SQ, D_MODEL), jnp.bfloat16),
            pltpu.VMEM((SQ, D_MODEL), jnp.bfloat16),
            pltpu.VMEM((SQ, D_MODEL), jnp.float32),
            pltpu.VMEM((SQ, D_MODEL), jnp.bfloat16),
            pltpu.VMEM((4, SQ // 2, HALF_D), jnp.bfloat16),
            pltpu.VMEM((4, SQ // 2, HALF_D), jnp.bfloat16),
            pltpu.VMEM((SQ, HALF_D), jnp.bfloat16),
            pltpu.VMEM((SQ, HALF_D), jnp.bfloat16),
            pltpu.SemaphoreType.DMA((4,)),
            pltpu.SemaphoreType.DMA((4,)),
            pltpu.SemaphoreType.DMA((4,)),
            pltpu.SemaphoreType.DMA((4,)),
            pltpu.SemaphoreType.DMA((4,)),
            pltpu.SemaphoreType.DMA((4,)),
            pltpu.SemaphoreType.DMA((4,)),
            pltpu.SemaphoreType.DMA((4,)),
        ],
        compiler_params=pltpu.CompilerParams(collective_id=0),
    )(xb, wq, kh, vh, wo)
    return out.reshape(1, SQ, D_MODEL)
